# Initial kernel scaffold; baseline (speedup 1.0000x reference)
import jax
import jax.numpy as jnp
from jax.experimental import pallas as pl


def kernel(hidden_states, Wr, W1, b1, W2, b2):
    raise NotImplementedError("write your pallas kernel here")


# speedup vs baseline: 5.7635x; 5.7635x over previous
import functools

import jax
import jax.numpy as jnp
from jax.experimental import pallas as pl
from jax.experimental.pallas import tpu as pltpu

_B, _S, _D = 1, 2048, 768
_FFN = 3072
_E = 64
_K = 2
_T = 128
_FB = 768
_C = _FFN // _FB
_MAXT = _S // _T
_NP = _S * _K
_TOT = _NP + _E * 8
_TOTP = _TOT + _T


def _ffn_kernel(off_ref, xg_ref, sc_ref, w1_ref, b1_ref, w2_ref, b2_ref,
                y_ref):
    e = pl.program_id(0)
    c = pl.program_id(1)
    start = off_ref[e]
    end = off_ref[e + 1]
    w1 = w1_ref[0]
    w2 = w2_ref[0]
    b1 = b1_ref[0]
    for t in range(_MAXT):
        @pl.when(start + t * _T < end)
        def _():
            s0 = pl.multiple_of(start + t * _T, 8)
            x = xg_ref[pl.ds(s0, _T), :]
            h = jnp.dot(x, w1, preferred_element_type=jnp.float32) + b1
            h = 0.5 * h * (1.0 + jax.lax.erf(h * 0.7071067811865476))
            yp = jnp.dot(h, w2, preferred_element_type=jnp.float32)

            @pl.when(c == 0)
            def _():
                y_ref[pl.ds(s0, _T), :] = yp

            @pl.when(c != 0)
            def _():
                y_ref[pl.ds(s0, _T), :] += yp

            @pl.when(c == _C - 1)
            def _():
                y_ref[pl.ds(s0, _T), :] = (
                    (y_ref[pl.ds(s0, _T), :] + b2_ref[0])
                    * sc_ref[pl.ds(s0, _T), :])


def _grouped_ffn(off, xg, sc2d, W1, b1r, W2, b2r):
    grid_spec = pltpu.PrefetchScalarGridSpec(
        num_scalar_prefetch=1,
        grid=(_E, _C),
        in_specs=[
            pl.BlockSpec((_TOTP, _D), lambda e, c, off: (0, 0)),
            pl.BlockSpec((_TOTP, 1), lambda e, c, off: (0, 0)),
            pl.BlockSpec((1, _D, _FB), lambda e, c, off: (e, 0, c)),
            pl.BlockSpec((1, 1, _FB), lambda e, c, off: (e, 0, c)),
            pl.BlockSpec((1, _FB, _D), lambda e, c, off: (e, c, 0)),
            pl.BlockSpec((1, 1, _D), lambda e, c, off: (e, 0, 0)),
        ],
        out_specs=pl.BlockSpec((_TOTP, _D), lambda e, c, off: (0, 0)),
    )
    return pl.pallas_call(
        _ffn_kernel,
        grid_spec=grid_spec,
        out_shape=jax.ShapeDtypeStruct((_TOTP, _D), jnp.float32),
        compiler_params=pltpu.CompilerParams(
            dimension_semantics=("arbitrary", "arbitrary")),
    )(off, xg, sc2d, W1, b1r, W2, b2r)


@jax.jit
def kernel(hidden_states, Wr, W1, b1, W2, b2):
    flat = hidden_states.reshape(_S, _D)
    logits = flat @ Wr.T
    top_vals, top_idx = jax.lax.top_k(logits, _K)
    probs = jax.nn.softmax(top_vals, axis=-1)

    eids = top_idx.reshape(-1)
    counts = jnp.bincount(eids, length=_E)
    cpad = ((counts + 7) // 8) * 8
    off = jnp.concatenate(
        [jnp.zeros((1,), jnp.int32),
         jnp.cumsum(cpad).astype(jnp.int32)])
    order = jnp.argsort(eids, stable=True)
    eids_sorted = eids[order]
    csum = jnp.cumsum(counts)
    rank = jnp.arange(_NP) - (csum[eids_sorted] - counts[eids_sorted])
    pos = off[eids_sorted] + rank

    sorted_pair = jnp.zeros((_TOTP,), jnp.int32).at[pos].set(
        order.astype(jnp.int32))
    scale = jnp.zeros((_TOTP,), jnp.float32).at[pos].set(
        probs.reshape(-1)[order])
    valid = jnp.zeros((_TOTP,), jnp.bool_).at[pos].set(True)
    tok = sorted_pair // _K
    xg = flat[tok]

    yg = _grouped_ffn(off, xg, scale[:, None], W1,
                      b1.reshape(_E, 1, _FFN), W2, b2.reshape(_E, 1, _D))

    yg = jnp.where(valid[:, None], yg, 0.0)
    out = jnp.zeros((_S, _D), jnp.float32).at[tok[:_TOT]].add(yg[:_TOT])
    return out.reshape(_B, _S, _D)


# speedup vs baseline: 5.8400x; 1.0133x over previous
import functools

import jax
import jax.numpy as jnp
from jax.experimental import pallas as pl
from jax.experimental.pallas import tpu as pltpu

_B, _S, _D = 1, 2048, 768
_FFN = 3072
_E = 64
_K = 2
_T = 128
_FB = 768
_C = _FFN // _FB
_MAXT = _S // _T
_NP = _S * _K
_TOT = _NP + _E * 8
_TOTP = _TOT + _T


def _ffn_kernel(off_ref, xg_ref, sc_ref, w1_ref, b1_ref, w2_ref, b2_ref,
                y_ref):
    e = pl.program_id(0)
    c = pl.program_id(1)
    start = off_ref[e]
    end = off_ref[e + 1]
    w1 = w1_ref[0]
    w2 = w2_ref[0]
    b1 = b1_ref[0]
    for t in range(_MAXT):
        @pl.when(start + t * _T < end)
        def _():
            s0 = pl.multiple_of(start + t * _T, 8)
            x = xg_ref[pl.ds(s0, _T), :]
            h = jnp.dot(x, w1, preferred_element_type=jnp.float32) + b1
            h = 0.5 * h * (1.0 + jax.lax.erf(h * 0.7071067811865476))
            yp = jnp.dot(h, w2, preferred_element_type=jnp.float32)

            @pl.when(c == 0)
            def _():
                y_ref[pl.ds(s0, _T), :] = yp

            @pl.when(c != 0)
            def _():
                y_ref[pl.ds(s0, _T), :] += yp

            @pl.when(c == _C - 1)
            def _():
                y_ref[pl.ds(s0, _T), :] = (
                    (y_ref[pl.ds(s0, _T), :] + b2_ref[0])
                    * sc_ref[pl.ds(s0, _T), :])


def _grouped_ffn(off, xg, sc2d, W1, b1r, W2, b2r):
    grid_spec = pltpu.PrefetchScalarGridSpec(
        num_scalar_prefetch=1,
        grid=(_E, _C),
        in_specs=[
            pl.BlockSpec((_TOTP, _D), lambda e, c, off: (0, 0)),
            pl.BlockSpec((_TOTP, 1), lambda e, c, off: (0, 0)),
            pl.BlockSpec((1, _D, _FB), lambda e, c, off: (e, 0, c)),
            pl.BlockSpec((1, 1, _FB), lambda e, c, off: (e, 0, c)),
            pl.BlockSpec((1, _FB, _D), lambda e, c, off: (e, c, 0)),
            pl.BlockSpec((1, 1, _D), lambda e, c, off: (e, 0, 0)),
        ],
        out_specs=pl.BlockSpec((_TOTP, _D), lambda e, c, off: (0, 0)),
    )
    return pl.pallas_call(
        _ffn_kernel,
        grid_spec=grid_spec,
        out_shape=jax.ShapeDtypeStruct((_TOTP, _D), jnp.float32),
        compiler_params=pltpu.CompilerParams(
            dimension_semantics=("arbitrary", "arbitrary")),
    )(off, xg, sc2d, W1, b1r, W2, b2r)


@jax.jit
def kernel(hidden_states, Wr, W1, b1, W2, b2):
    flat = hidden_states.reshape(_S, _D)
    logits = flat @ Wr.T
    top_vals, top_idx = jax.lax.top_k(logits, _K)
    probs = jax.nn.softmax(top_vals, axis=-1)

    eids = top_idx.reshape(-1)
    onehot = (eids[:, None] == jnp.arange(_E)[None, :]).astype(jnp.int32)
    counts = onehot.sum(axis=0)
    cpad = ((counts + 7) // 8) * 8
    off = jnp.concatenate(
        [jnp.zeros((1,), jnp.int32),
         jnp.cumsum(cpad).astype(jnp.int32)])
    ranks = jnp.cumsum(onehot, axis=0)
    rank = jnp.take_along_axis(ranks, eids[:, None], axis=1)[:, 0] - 1
    pos = off[eids] + rank

    slot = jnp.arange(_TOTP, dtype=jnp.int32)
    e_of_s = jnp.minimum(
        (slot[:, None] >= off[None, 1:]).sum(axis=1), _E - 1)
    r = slot - off[e_of_s]
    valid = (r >= 0) & (r < counts[e_of_s]) & (slot < off[_E])
    c0 = jnp.cumsum(counts) - counts
    order = jnp.argsort(eids)
    src = jnp.minimum(c0[e_of_s] + jnp.maximum(r, 0), _NP - 1)
    sorted_pair = jnp.where(valid, order[src], 0)
    scale = jnp.where(valid, probs.reshape(-1)[order][src], 0.0)
    tok = sorted_pair // _K
    xg = flat[tok]

    yg = _grouped_ffn(off, xg, scale[:, None], W1,
                      b1.reshape(_E, 1, _FFN), W2, b2.reshape(_E, 1, _D))

    out = yg[pos].reshape(_S, _K, _D).sum(axis=1)
    return out.reshape(_B, _S, _D)
